# baseline (speedup 1.0000x reference)
import functools

import jax
import jax.numpy as jnp
from jax import lax
from jax.experimental import pallas as pl
from jax.experimental.pallas import tpu as pltpu
from jax.experimental.pallas import tpu_sc as plsc

N = 10000
NP = 10240
E = 320000
E_PAD = 327680
D_IN = 128
D_H = 256
D_OUT = 64
N_EXP = 8
ROWB = 1024
NC, NS = 2, 16
RDEG = E_PAD // NS // 128
RPROP = E_PAD // (NC * NS) // 128
NSLICE = NP // NS


def _sc_mesh():
    return plsc.VectorSubcoreMesh(
        core_axis_name="c", subcore_axis_name="s", num_cores=NC, num_subcores=NS)


_SC_PARAMS = pltpu.CompilerParams(needs_layout_passes=False)


def _rsqrt_newton(d):
    i = lax.bitcast_convert_type(d, jnp.int32)
    i = 0x5F3759DF - lax.shift_right_logical(i, 1)
    y = lax.bitcast_convert_type(i, jnp.float32)
    for _ in range(3):
        y = y * (1.5 - 0.5 * d * y * y)
    return y


def _scale_rows(rows_v, w_v, nrows):
    def body(r, cc):
        wv = plsc.load_gather(w_v, [jnp.zeros((16,), jnp.int32) + r])
        for v in range(8):
            rows_v[r, pl.ds(v * 16, 16)] = rows_v[r, pl.ds(v * 16, 16)] * wv
        return cc

    lax.fori_loop(0, nrows, body, 0)


def _pipelined_bursts(compute_iw, gather_src, rows, scat_dst, w_bufs, gsems, ssems):
    scat = [None, None]
    gath = [None, None]
    compute_iw(0, 0)
    gath[0] = pltpu.async_copy(gather_src(0, 0), rows[0], gsems[0])
    for jj in range(8):
        b = jj % 2
        nb = (jj + 1) % 2
        if jj < 7:
            compute_iw(jj + 1, nb)
            if scat[nb] is not None:
                scat[nb].wait()
                scat[nb] = None
            gath[nb] = pltpu.async_copy(gather_src(jj + 1, nb), rows[nb], gsems[nb])
        gath[b].wait()
        _scale_rows(rows[b], w_bufs[b], 128)
        if scat[b] is not None:
            scat[b].wait()
        scat[b] = pltpu.async_copy(rows[b], scat_dst(jj), ssems[b], add=True)
    scat[0].wait()
    scat[1].wait()


def _prop1_body(srcp, dstp, xp, z2d, z1d,
                ppart, dis_out,
                deg_sh, p_sh,
                swin_v, dwin_v, dis_v, ones_v, w0_v, w1_v, rows0_v, rows1_v,
                gsem0, gsem1, ssem0, ssem1):
    c = lax.axis_index("c")
    s = lax.axis_index("s")
    wid = c * NS + s
    rows = [rows0_v, rows1_v]
    w_bufs = [w0_v, w1_v]
    gsems = [gsem0, gsem1]
    ssems = [ssem0, ssem1]

    pltpu.sync_copy(z2d, p_sh.at[pl.ds(s * NSLICE, NSLICE), :])

    @pl.when(s == 0)
    def _():
        pltpu.sync_copy(z1d, deg_sh)

    for v in range(8):
        ones_v[pl.ds(v * 16, 16)] = jnp.full((16,), 1.0, jnp.float32)
    plsc.subcore_barrier()

    def deg_win(wi, carry):
        pltpu.sync_copy(dstp.at[pl.ds(s * RDEG + wi * 8, 8), :], dwin_v)
        descs = [pltpu.async_copy(ones_v, deg_sh.at[dwin_v.at[jj]], gsem0,
                                  add=True) for jj in range(8)]
        for d in descs:
            d.wait()
        return carry

    lax.fori_loop(0, RDEG // 8, deg_win, 0)
    plsc.subcore_barrier()

    pltpu.sync_copy(deg_sh, dis_v)

    def dis_body(j, carry):
        d = jnp.maximum(dis_v[pl.ds(j * 16, 16)], 1.0)
        dis_v[pl.ds(j * 16, 16)] = _rsqrt_newton(d)
        return carry

    lax.fori_loop(0, NP // 16, dis_body, 0)

    def prop_win(wi, carry):
        pltpu.sync_copy(srcp.at[pl.ds(wid * RPROP + wi * 8, 8), :], swin_v)
        pltpu.sync_copy(dstp.at[pl.ds(wid * RPROP + wi * 8, 8), :], dwin_v)

        def compute_iw(jj, b):
            wb = w_bufs[b]
            for g in range(8):
                sv = swin_v[jj, pl.ds(g * 16, 16)]
                dv = dwin_v[jj, pl.ds(g * 16, 16)]
                wb[pl.ds(g * 16, 16)] = (plsc.load_gather(dis_v, [sv]) *
                                         plsc.load_gather(dis_v, [dv]))

        _pipelined_bursts(compute_iw,
                          lambda jj, b: xp.at[swin_v.at[jj]],
                          rows,
                          lambda jj: p_sh.at[dwin_v.at[jj]],
                          w_bufs, gsems, ssems)
        return carry

    lax.fori_loop(0, RPROP // 8, prop_win, 0)
    plsc.subcore_barrier()

    def wb(chunk, carry):
        base = s * NSLICE + chunk * 128
        pltpu.sync_copy(p_sh.at[pl.ds(base, 128), :], rows0_v)

        def wcopy(i, cc):
            w0_v[pl.ds(i * 16, 16)] = dis_v[pl.ds(base + i * 16, 16)]
            return cc

        lax.fori_loop(0, 8, wcopy, 0)
        _scale_rows(rows0_v, w0_v, 128)
        pltpu.sync_copy(rows0_v, ppart.at[c, pl.ds(base, 128), :])
        return carry

    lax.fori_loop(0, NSLICE // 128, wb, 0)

    @pl.when((s == 0) & (c == 0))
    def _():
        pltpu.sync_copy(dis_v, dis_out)


def _prop1(srcp, dstp, xp):
    z2d = jnp.zeros((NSLICE, 128), jnp.float32)
    z1d = jnp.zeros((NP,), jnp.float32)
    return pl.kernel(
        _prop1_body,
        out_type=(
            jax.ShapeDtypeStruct((NC, NP, 128), jnp.float32),
            jax.ShapeDtypeStruct((NP,), jnp.float32),
        ),
        mesh=_sc_mesh(),
        compiler_params=_SC_PARAMS,
        scratch_types=[
            pltpu.VMEM_SHARED((NP,), jnp.float32),
            pltpu.VMEM_SHARED((NP, 128), jnp.float32),
            pltpu.VMEM((8, 128), jnp.int32),
            pltpu.VMEM((8, 128), jnp.int32),
            pltpu.VMEM((NP,), jnp.float32),
            pltpu.VMEM((128,), jnp.float32),
            pltpu.VMEM((128,), jnp.float32),
            pltpu.VMEM((128,), jnp.float32),
            pltpu.VMEM((128, 128), jnp.float32),
            pltpu.VMEM((128, 128), jnp.float32),
            pltpu.SemaphoreType.DMA,
            pltpu.SemaphoreType.DMA,
            pltpu.SemaphoreType.DMA,
            pltpu.SemaphoreType.DMA,
        ],
    )(srcp, dstp, xp, z2d, z1d)


RC = E_PAD // NS // 64


def _combine_rows(rb1, rb2, rowsf, w1_v, w2_v):
    def body(r, cc):
        wv1 = plsc.load_gather(w1_v, [jnp.zeros((16,), jnp.int32) + r])
        wv2 = plsc.load_gather(w2_v, [jnp.zeros((16,), jnp.int32) + r])
        for v in range(8):
            sl = pl.ds(v * 16, 16)
            rowsf[r, sl] = rb1[r, sl] * wv1 + rb2[r, sl] * wv2
        return cc

    lax.fori_loop(0, 64, body, 0)


def _combine_body(srcp, dstp, qb, gd1p, gd2p, z2d,
                  outc,
                  out_sh,
                  swin_v, dwin_v, gd1_v, gd2_v, w1_v, w2_v, idx1_v, idx2_v,
                  rb1_v, rb2_v, rowsf_v, gsem0, gsem1, ssem0):
    c = lax.axis_index("c")
    s = lax.axis_index("s")
    cbase = c * (N_EXP * NP)

    pltpu.sync_copy(z2d, out_sh.at[pl.ds(s * NSLICE, NSLICE), :])
    pltpu.sync_copy(gd1p, gd1_v)
    pltpu.sync_copy(gd2p, gd2_v)
    plsc.subcore_barrier()

    def win_body(wi, carry):
        pltpu.sync_copy(srcp.at[pl.ds(s * RC + wi * 8, 8), :], swin_v)
        pltpu.sync_copy(dstp.at[pl.ds(s * RC + wi * 8, 8), :], dwin_v)

        def compute_iw(jj):
            for g in range(4):
                sl = pl.ds(g * 16, 16)
                sv = swin_v[jj, sl]
                dv = dwin_v[jj, sl]
                v1 = plsc.load_gather(gd1_v, [dv])
                e1 = v1 & 7
                w1_v[sl] = lax.bitcast_convert_type(v1 - e1, jnp.float32)
                idx1_v[sl] = cbase + e1 * NP + sv
                v2 = plsc.load_gather(gd2_v, [dv])
                e2 = v2 & 7
                w2_v[sl] = lax.bitcast_convert_type(v2 - e2, jnp.float32)
                idx2_v[sl] = cbase + e2 * NP + sv

        scat = [None]
        compute_iw(0)
        g1 = pltpu.async_copy(qb.at[idx1_v], rb1_v, gsem0)
        g2 = pltpu.async_copy(qb.at[idx2_v], rb2_v, gsem1)
        for jj in range(8):
            g1.wait()
            g2.wait()
            if scat[0] is not None:
                scat[0].wait()
            _combine_rows(rb1_v, rb2_v, rowsf_v, w1_v, w2_v)
            if jj < 7:
                compute_iw(jj + 1)
                g1 = pltpu.async_copy(qb.at[idx1_v], rb1_v, gsem0)
                g2 = pltpu.async_copy(qb.at[idx2_v], rb2_v, gsem1)
            scat[0] = pltpu.async_copy(rowsf_v, out_sh.at[dwin_v.at[jj]],
                                       ssem0, add=True)
        scat[0].wait()
        return carry

    lax.fori_loop(0, RC // 8, win_body, 0)

    plsc.subcore_barrier()
    pltpu.sync_copy(out_sh.at[pl.ds(s * NSLICE, NSLICE), :],
                    outc.at[c, pl.ds(s * NSLICE, NSLICE), :])


def _combine(srcp64, dstp64, QB, gd1p, gd2p):
    z2d = jnp.zeros((NSLICE, 128), jnp.float32)
    return pl.kernel(
        _combine_body,
        out_type=jax.ShapeDtypeStruct((NC, NP, 128), jnp.float32),
        mesh=_sc_mesh(),
        compiler_params=_SC_PARAMS,
        scratch_types=[
            pltpu.VMEM_SHARED((NP, 128), jnp.float32),
            pltpu.VMEM((8, 64), jnp.int32),
            pltpu.VMEM((8, 64), jnp.int32),
            pltpu.VMEM((NP,), jnp.int32),
            pltpu.VMEM((NP,), jnp.int32),
            pltpu.VMEM((64,), jnp.float32),
            pltpu.VMEM((64,), jnp.float32),
            pltpu.VMEM((64,), jnp.int32),
            pltpu.VMEM((64,), jnp.int32),
            pltpu.VMEM((64, 128), jnp.float32),
            pltpu.VMEM((64, 128), jnp.float32),
            pltpu.VMEM((64, 128), jnp.float32),
            pltpu.SemaphoreType.DMA,
            pltpu.SemaphoreType.DMA,
            pltpu.SemaphoreType.DMA,
        ],
    )(srcp64, dstp64, QB, gd1p, gd2p, z2d)


def _gate_body(wgT_ref, xT_ref, dis_ref, gd1_ref, gd2_ref):
    logits = jnp.dot(wgT_ref[...], xT_ref[...], preferred_element_type=jnp.float32)
    m = jnp.max(logits, axis=0, keepdims=True)
    ex = jnp.exp(logits - m)
    gates = ex / jnp.sum(ex, axis=0, keepdims=True)
    iota = lax.broadcasted_iota(jnp.int32, gates.shape, 0)
    v1 = jnp.max(gates, axis=0, keepdims=True)
    i1 = jnp.min(jnp.where(gates == v1, iota, N_EXP), axis=0, keepdims=True)
    masked = jnp.where(iota == i1, -1.0, gates)
    v2 = jnp.max(masked, axis=0, keepdims=True)
    i2 = jnp.min(jnp.where(masked == v2, iota, N_EXP), axis=0, keepdims=True)
    sd = dis_ref[...] / (v1 + v2 + 1e-9)
    b1 = lax.bitcast_convert_type(v1 * sd, jnp.int32)
    b2 = lax.bitcast_convert_type(v2 * sd, jnp.int32)
    gd1_ref[...] = (b1 & ~7) | i1
    gd2_ref[...] = (b2 & ~7) | i2


def _gating(xT_pad, WgT, dis):
    return pl.pallas_call(
        _gate_body,
        out_shape=(
            jax.ShapeDtypeStruct((1, NP), jnp.int32),
            jax.ShapeDtypeStruct((1, NP), jnp.int32),
        ),
    )(WgT, xT_pad, dis)


def _experts_body(p0_ref, p1_ref, w1_ref, w2_ref, q_ref):
    p = p0_ref[...] + p1_ref[...]
    h = jnp.maximum(jnp.dot(p, w1_ref[0], preferred_element_type=jnp.float32), 0.0)
    q = jnp.dot(h, w2_ref[0], preferred_element_type=jnp.float32)
    q_ref[0, 0] = q[:, :128]
    q_ref[1, 0] = q[:, 128:]


def _experts(P0, P1, W1, W2):
    grid = (N_EXP, NP // ROWB)
    return pl.pallas_call(
        _experts_body,
        grid=grid,
        in_specs=[
            pl.BlockSpec((ROWB, D_IN), lambda e, r: (r, 0)),
            pl.BlockSpec((ROWB, D_IN), lambda e, r: (r, 0)),
            pl.BlockSpec((1, D_IN, D_H), lambda e, r: (e, 0, 0)),
            pl.BlockSpec((1, D_H, D_H), lambda e, r: (e, 0, 0)),
        ],
        out_specs=pl.BlockSpec((2, 1, ROWB, 128), lambda e, r: (0, e, r, 0)),
        out_shape=jax.ShapeDtypeStruct((2, N_EXP, NP, 128), jnp.float32),
    )(P0, P1, W1, W2)


def _mlp_body(o0_ref, o1_ref, wa_ref, wb_ref, b1_ref, w2_ref, b2_ref, y_ref):
    h = jnp.dot(o0_ref[...], wa_ref[...], preferred_element_type=jnp.float32)
    h += jnp.dot(o1_ref[...], wb_ref[...], preferred_element_type=jnp.float32)
    h = jnp.maximum(h + b1_ref[...], 0.0)
    y_ref[...] = jnp.dot(h, w2_ref[...], preferred_element_type=jnp.float32) + b2_ref[...]


def _mlp(out0, out1, Wm1, bm1, Wm2, bm2):
    grid = (NP // ROWB,)
    return pl.pallas_call(
        _mlp_body,
        grid=grid,
        in_specs=[
            pl.BlockSpec((ROWB, 128), lambda r: (r, 0)),
            pl.BlockSpec((ROWB, 128), lambda r: (r, 0)),
            pl.BlockSpec((128, D_H), lambda r: (0, 0)),
            pl.BlockSpec((128, D_H), lambda r: (0, 0)),
            pl.BlockSpec((1, D_H), lambda r: (0, 0)),
            pl.BlockSpec((D_H, D_OUT), lambda r: (0, 0)),
            pl.BlockSpec((1, D_OUT), lambda r: (0, 0)),
        ],
        out_specs=pl.BlockSpec((ROWB, D_OUT), lambda r: (r, 0)),
        out_shape=jax.ShapeDtypeStruct((NP, D_OUT), jnp.float32),
    )(out0, out1, Wm1[:128], Wm1[128:], bm1[None, :], Wm2, bm2[None, :])


def kernel(x, edge_index, batch, Wg, W1, W2, Wm1, bm1, Wm2, bm2):
    src = edge_index[0]
    dst = edge_index[1]
    xp = jnp.pad(x, ((0, NP - N), (0, 0)))

    pad = jnp.full((E_PAD - E,), NP - 1, jnp.int32)
    srcp = jnp.concatenate([src, pad]).reshape(E_PAD // 128, 128)
    dstp = jnp.concatenate([dst, pad]).reshape(E_PAD // 128, 128)

    Ppart, dis = _prop1(srcp, dstp, xp)

    gd1p, gd2p = _gating(xp.T, Wg.T, dis[None, :])

    QB = _experts(Ppart[0], Ppart[1], W1, W2).reshape(2 * N_EXP * NP, 128)

    outc = _combine(srcp.reshape(E_PAD // 64, 64),
                    dstp.reshape(E_PAD // 64, 64), QB, gd1p[0], gd2p[0])

    y = _mlp(outc[0], outc[1], Wm1, bm1, Wm2, bm2)
    return y[:N]


# speedup vs baseline: 1.3677x; 1.3677x over previous
import functools

import jax
import jax.numpy as jnp
from jax import lax
from jax.experimental import pallas as pl
from jax.experimental.pallas import tpu as pltpu
from jax.experimental.pallas import tpu_sc as plsc

N = 10000
NP = 10240
E = 320000
E_PAD = 327680
D_IN = 128
D_H = 256
D_OUT = 64
N_EXP = 8
ROWB = 1024
NC, NS = 2, 16
RDEG = E_PAD // NS // 128
RPROP = E_PAD // (NC * NS) // 128
NSLICE = NP // NS


def _sc_mesh():
    return plsc.VectorSubcoreMesh(
        core_axis_name="c", subcore_axis_name="s", num_cores=NC, num_subcores=NS)


_SC_PARAMS = pltpu.CompilerParams(needs_layout_passes=False)


def _rsqrt_newton(d):
    i = lax.bitcast_convert_type(d, jnp.int32)
    i = 0x5F3759DF - lax.shift_right_logical(i, 1)
    y = lax.bitcast_convert_type(i, jnp.float32)
    for _ in range(3):
        y = y * (1.5 - 0.5 * d * y * y)
    return y


def _scale_rows(rows_v, w_v, nrows):
    def body(r2, cc):
        ra = r2 * 2
        wva = plsc.load_gather(w_v, [jnp.zeros((16,), jnp.int32) + ra])
        wvb = plsc.load_gather(w_v, [jnp.zeros((16,), jnp.int32) + (ra + 1)])
        for v in range(8):
            sl = pl.ds(v * 16, 16)
            rows_v[ra, sl] = rows_v[ra, sl] * wva
            rows_v[ra + 1, sl] = rows_v[ra + 1, sl] * wvb
        return cc

    lax.fori_loop(0, nrows // 2, body, 0)


def _pipelined_bursts(compute_iw, gather_src, rows, scat_dst, w_bufs, gsems, ssems):
    scat = [None, None]
    gath = [None, None]
    compute_iw(0, 0)
    gath[0] = pltpu.async_copy(gather_src(0, 0), rows[0], gsems[0])
    for jj in range(8):
        b = jj % 2
        nb = (jj + 1) % 2
        if jj < 7:
            compute_iw(jj + 1, nb)
            if scat[nb] is not None:
                scat[nb].wait()
                scat[nb] = None
            gath[nb] = pltpu.async_copy(gather_src(jj + 1, nb), rows[nb], gsems[nb])
        gath[b].wait()
        _scale_rows(rows[b], w_bufs[b], 128)
        if scat[b] is not None:
            scat[b].wait()
        scat[b] = pltpu.async_copy(rows[b], scat_dst(jj), ssems[b], add=True)
    scat[0].wait()
    scat[1].wait()


def _prop1_body(srcp, dstp, xp, z2d, z1d,
                ppart, dis_out,
                deg_sh, p_sh,
                swin_v, dwin_v, dis_v, ones_v, w0_v, w1_v, rows0_v, rows1_v,
                gsem0, gsem1, ssem0, ssem1):
    c = lax.axis_index("c")
    s = lax.axis_index("s")
    wid = c * NS + s
    rows = [rows0_v, rows1_v]
    w_bufs = [w0_v, w1_v]
    gsems = [gsem0, gsem1]
    ssems = [ssem0, ssem1]

    pltpu.sync_copy(z2d, p_sh.at[pl.ds(s * NSLICE, NSLICE), :])

    @pl.when(s == 0)
    def _():
        pltpu.sync_copy(z1d, deg_sh)

    for v in range(8):
        ones_v[pl.ds(v * 16, 16)] = jnp.full((16,), 1.0, jnp.float32)
    plsc.subcore_barrier()

    def deg_win(wi, carry):
        pltpu.sync_copy(dstp.at[pl.ds(s * RDEG + wi * 8, 8), :], dwin_v)
        descs = [pltpu.async_copy(ones_v, deg_sh.at[dwin_v.at[jj]], gsem0,
                                  add=True) for jj in range(8)]
        for d in descs:
            d.wait()
        return carry

    lax.fori_loop(0, RDEG // 8, deg_win, 0)
    plsc.subcore_barrier()

    pltpu.sync_copy(deg_sh, dis_v)

    def dis_body(j, carry):
        d = jnp.maximum(dis_v[pl.ds(j * 16, 16)], 1.0)
        dis_v[pl.ds(j * 16, 16)] = _rsqrt_newton(d)
        return carry

    lax.fori_loop(0, NP // 16, dis_body, 0)

    def prop_win(wi, carry):
        pltpu.sync_copy(srcp.at[pl.ds(wid * RPROP + wi * 8, 8), :], swin_v)
        pltpu.sync_copy(dstp.at[pl.ds(wid * RPROP + wi * 8, 8), :], dwin_v)

        def compute_iw(jj, b):
            wb = w_bufs[b]
            for g in range(8):
                sv = swin_v[jj, pl.ds(g * 16, 16)]
                dv = dwin_v[jj, pl.ds(g * 16, 16)]
                wb[pl.ds(g * 16, 16)] = (plsc.load_gather(dis_v, [sv]) *
                                         plsc.load_gather(dis_v, [dv]))

        _pipelined_bursts(compute_iw,
                          lambda jj, b: xp.at[swin_v.at[jj]],
                          rows,
                          lambda jj: p_sh.at[dwin_v.at[jj]],
                          w_bufs, gsems, ssems)
        return carry

    lax.fori_loop(0, RPROP // 8, prop_win, 0)
    plsc.subcore_barrier()

    def wb(chunk, carry):
        base = s * NSLICE + chunk * 128
        pltpu.sync_copy(p_sh.at[pl.ds(base, 128), :], rows0_v)

        def wcopy(i, cc):
            w0_v[pl.ds(i * 16, 16)] = dis_v[pl.ds(base + i * 16, 16)]
            return cc

        lax.fori_loop(0, 8, wcopy, 0)
        _scale_rows(rows0_v, w0_v, 128)
        pltpu.sync_copy(rows0_v, ppart.at[c, pl.ds(base, 128), :])
        return carry

    lax.fori_loop(0, NSLICE // 128, wb, 0)

    @pl.when((s == 0) & (c == 0))
    def _():
        pltpu.sync_copy(dis_v, dis_out)


def _prop1(srcp, dstp, xp):
    z2d = jnp.zeros((NSLICE, 128), jnp.float32)
    z1d = jnp.zeros((NP,), jnp.float32)
    return pl.kernel(
        _prop1_body,
        out_type=(
            jax.ShapeDtypeStruct((NC, NP, 128), jnp.float32),
            jax.ShapeDtypeStruct((NP,), jnp.float32),
        ),
        mesh=_sc_mesh(),
        compiler_params=_SC_PARAMS,
        scratch_types=[
            pltpu.VMEM_SHARED((NP,), jnp.float32),
            pltpu.VMEM_SHARED((NP, 128), jnp.float32),
            pltpu.VMEM((8, 128), jnp.int32),
            pltpu.VMEM((8, 128), jnp.int32),
            pltpu.VMEM((NP,), jnp.float32),
            pltpu.VMEM((128,), jnp.float32),
            pltpu.VMEM((128,), jnp.float32),
            pltpu.VMEM((128,), jnp.float32),
            pltpu.VMEM((128, 128), jnp.float32),
            pltpu.VMEM((128, 128), jnp.float32),
            pltpu.SemaphoreType.DMA,
            pltpu.SemaphoreType.DMA,
            pltpu.SemaphoreType.DMA,
            pltpu.SemaphoreType.DMA,
        ],
    )(srcp, dstp, xp, z2d, z1d)


def _combine_body(srcp, dstp, qb, gd1p, gd2p, z2d,
                  outc,
                  out_sh,
                  swin_v, dwin_v, gdp_v, w0_v, w1_v, idx0_v, idx1_v,
                  rows0_v, rows1_v, gsem0, gsem1, ssem0, ssem1):
    c = lax.axis_index("c")
    s = lax.axis_index("s")
    cbase = c * (N_EXP * NP)
    rows = [rows0_v, rows1_v]
    w_bufs = [w0_v, w1_v]
    idx_bufs = [idx0_v, idx1_v]
    gsems = [gsem0, gsem1]
    ssems = [ssem0, ssem1]

    pltpu.sync_copy(z2d, out_sh.at[pl.ds(s * NSLICE, NSLICE), :])
    plsc.subcore_barrier()

    for k in range(2):
        pltpu.sync_copy(gd1p if k == 0 else gd2p, gdp_v)

        def win_body(wi, carry):
            pltpu.sync_copy(srcp.at[pl.ds(s * RDEG + wi * 8, 8), :], swin_v)
            pltpu.sync_copy(dstp.at[pl.ds(s * RDEG + wi * 8, 8), :], dwin_v)

            def compute_iw(jj, b):
                for g in range(8):
                    sl = pl.ds(g * 16, 16)
                    sv = swin_v[jj, sl]
                    dv = dwin_v[jj, sl]
                    v = plsc.load_gather(gdp_v, [dv])
                    ek = v & 7
                    w_bufs[b][sl] = lax.bitcast_convert_type(v - ek, jnp.float32)
                    idx_bufs[b][sl] = cbase + ek * NP + sv

            _pipelined_bursts(compute_iw,
                              lambda jj, b: qb.at[idx_bufs[b]],
                              rows,
                              lambda jj: out_sh.at[dwin_v.at[jj]],
                              w_bufs, gsems, ssems)
            return carry

        lax.fori_loop(0, RDEG // 8, win_body, 0)

    plsc.subcore_barrier()
    pltpu.sync_copy(out_sh.at[pl.ds(s * NSLICE, NSLICE), :],
                    outc.at[c, pl.ds(s * NSLICE, NSLICE), :])


def _combine(srcp, dstp, QB, gd1p, gd2p):
    z2d = jnp.zeros((NSLICE, 128), jnp.float32)
    return pl.kernel(
        _combine_body,
        out_type=jax.ShapeDtypeStruct((NC, NP, 128), jnp.float32),
        mesh=_sc_mesh(),
        compiler_params=_SC_PARAMS,
        scratch_types=[
            pltpu.VMEM_SHARED((NP, 128), jnp.float32),
            pltpu.VMEM((8, 128), jnp.int32),
            pltpu.VMEM((8, 128), jnp.int32),
            pltpu.VMEM((NP,), jnp.int32),
            pltpu.VMEM((128,), jnp.float32),
            pltpu.VMEM((128,), jnp.float32),
            pltpu.VMEM((128,), jnp.int32),
            pltpu.VMEM((128,), jnp.int32),
            pltpu.VMEM((128, 128), jnp.float32),
            pltpu.VMEM((128, 128), jnp.float32),
            pltpu.SemaphoreType.DMA,
            pltpu.SemaphoreType.DMA,
            pltpu.SemaphoreType.DMA,
            pltpu.SemaphoreType.DMA,
        ],
    )(srcp, dstp, QB, gd1p, gd2p, z2d)


def _gate_body(wgT_ref, xT_ref, dis_ref, gd1_ref, gd2_ref):
    logits = jnp.dot(wgT_ref[...], xT_ref[...], preferred_element_type=jnp.float32)
    m = jnp.max(logits, axis=0, keepdims=True)
    ex = jnp.exp(logits - m)
    gates = ex / jnp.sum(ex, axis=0, keepdims=True)
    iota = lax.broadcasted_iota(jnp.int32, gates.shape, 0)
    v1 = jnp.max(gates, axis=0, keepdims=True)
    i1 = jnp.min(jnp.where(gates == v1, iota, N_EXP), axis=0, keepdims=True)
    masked = jnp.where(iota == i1, -1.0, gates)
    v2 = jnp.max(masked, axis=0, keepdims=True)
    i2 = jnp.min(jnp.where(masked == v2, iota, N_EXP), axis=0, keepdims=True)
    sd = dis_ref[...] / (v1 + v2 + 1e-9)
    b1 = lax.bitcast_convert_type(v1 * sd, jnp.int32)
    b2 = lax.bitcast_convert_type(v2 * sd, jnp.int32)
    gd1_ref[...] = (b1 & ~7) | i1
    gd2_ref[...] = (b2 & ~7) | i2


def _gating(xT_pad, WgT, dis):
    return pl.pallas_call(
        _gate_body,
        out_shape=(
            jax.ShapeDtypeStruct((1, NP), jnp.int32),
            jax.ShapeDtypeStruct((1, NP), jnp.int32),
        ),
    )(WgT, xT_pad, dis)


def _experts_body(p0_ref, p1_ref, w1_ref, w2_ref, q_ref):
    p = p0_ref[...] + p1_ref[...]
    h = jnp.maximum(jnp.dot(p, w1_ref[0], preferred_element_type=jnp.float32), 0.0)
    q = jnp.dot(h, w2_ref[0], preferred_element_type=jnp.float32)
    q_ref[0, 0] = q[:, :128]
    q_ref[1, 0] = q[:, 128:]


def _experts(P0, P1, W1, W2):
    grid = (N_EXP, NP // ROWB)
    return pl.pallas_call(
        _experts_body,
        grid=grid,
        in_specs=[
            pl.BlockSpec((ROWB, D_IN), lambda e, r: (r, 0)),
            pl.BlockSpec((ROWB, D_IN), lambda e, r: (r, 0)),
            pl.BlockSpec((1, D_IN, D_H), lambda e, r: (e, 0, 0)),
            pl.BlockSpec((1, D_H, D_H), lambda e, r: (e, 0, 0)),
        ],
        out_specs=pl.BlockSpec((2, 1, ROWB, 128), lambda e, r: (0, e, r, 0)),
        out_shape=jax.ShapeDtypeStruct((2, N_EXP, NP, 128), jnp.float32),
    )(P0, P1, W1, W2)


def _mlp_body(o0_ref, o1_ref, wa_ref, wb_ref, b1_ref, w2_ref, b2_ref, y_ref):
    h = jnp.dot(o0_ref[...], wa_ref[...], preferred_element_type=jnp.float32)
    h += jnp.dot(o1_ref[...], wb_ref[...], preferred_element_type=jnp.float32)
    h = jnp.maximum(h + b1_ref[...], 0.0)
    y_ref[...] = jnp.dot(h, w2_ref[...], preferred_element_type=jnp.float32) + b2_ref[...]


def _mlp(out0, out1, Wm1, bm1, Wm2, bm2):
    grid = (NP // ROWB,)
    return pl.pallas_call(
        _mlp_body,
        grid=grid,
        in_specs=[
            pl.BlockSpec((ROWB, 128), lambda r: (r, 0)),
            pl.BlockSpec((ROWB, 128), lambda r: (r, 0)),
            pl.BlockSpec((128, D_H), lambda r: (0, 0)),
            pl.BlockSpec((128, D_H), lambda r: (0, 0)),
            pl.BlockSpec((1, D_H), lambda r: (0, 0)),
            pl.BlockSpec((D_H, D_OUT), lambda r: (0, 0)),
            pl.BlockSpec((1, D_OUT), lambda r: (0, 0)),
        ],
        out_specs=pl.BlockSpec((ROWB, D_OUT), lambda r: (r, 0)),
        out_shape=jax.ShapeDtypeStruct((NP, D_OUT), jnp.float32),
    )(out0, out1, Wm1[:128], Wm1[128:], bm1[None, :], Wm2, bm2[None, :])


def kernel(x, edge_index, batch, Wg, W1, W2, Wm1, bm1, Wm2, bm2):
    src = edge_index[0]
    dst = edge_index[1]
    xp = jnp.pad(x, ((0, NP - N), (0, 0)))

    pad = jnp.full((E_PAD - E,), NP - 1, jnp.int32)
    srcp = jnp.concatenate([src, pad]).reshape(E_PAD // 128, 128)
    dstp = jnp.concatenate([dst, pad]).reshape(E_PAD // 128, 128)

    Ppart, dis = _prop1(srcp, dstp, xp)

    gd1p, gd2p = _gating(xp.T, Wg.T, dis[None, :])

    QB = _experts(Ppart[0], Ppart[1], W1, W2).reshape(2 * N_EXP * NP, 128)

    outc = _combine(srcp, dstp, QB, gd1p[0], gd2p[0])

    y = _mlp(outc[0], outc[1], Wm1, bm1, Wm2, bm2)
    return y[:N]


# speedup vs baseline: 1.3909x; 1.0170x over previous
import functools

import jax
import jax.numpy as jnp
from jax import lax
from jax.experimental import pallas as pl
from jax.experimental.pallas import tpu as pltpu
from jax.experimental.pallas import tpu_sc as plsc

N = 10000
NP = 10240
E = 320000
E_PAD = 327680
D_IN = 128
D_H = 256
D_OUT = 64
N_EXP = 8
ROWB = 1024
NC, NS = 2, 16
RDEG = E_PAD // NS // 128
RPROP = E_PAD // (NC * NS) // 128
NSLICE = NP // NS


def _sc_mesh():
    return plsc.VectorSubcoreMesh(
        core_axis_name="c", subcore_axis_name="s", num_cores=NC, num_subcores=NS)


_SC_PARAMS = pltpu.CompilerParams(needs_layout_passes=False)


def _rsqrt_newton(d):
    i = lax.bitcast_convert_type(d, jnp.int32)
    i = 0x5F3759DF - lax.shift_right_logical(i, 1)
    y = lax.bitcast_convert_type(i, jnp.float32)
    for _ in range(3):
        y = y * (1.5 - 0.5 * d * y * y)
    return y


def _scale_rows(rows_v, w_v, nrows):
    def body(r2, cc):
        ra = r2 * 2
        wva = plsc.load_gather(w_v, [jnp.zeros((16,), jnp.int32) + ra])
        wvb = plsc.load_gather(w_v, [jnp.zeros((16,), jnp.int32) + (ra + 1)])
        for v in range(8):
            sl = pl.ds(v * 16, 16)
            rows_v[ra, sl] = rows_v[ra, sl] * wva
            rows_v[ra + 1, sl] = rows_v[ra + 1, sl] * wvb
        return cc

    lax.fori_loop(0, nrows // 2, body, 0)


def _pipelined_bursts(compute_iw, gather_src, rows, scat_dst, w_bufs, gsems, ssems):
    scat = [None, None]
    gath = [None, None]
    compute_iw(0, 0)
    gath[0] = pltpu.async_copy(gather_src(0, 0), rows[0], gsems[0])
    for jj in range(8):
        b = jj % 2
        nb = (jj + 1) % 2
        if jj < 7:
            compute_iw(jj + 1, nb)
            if scat[nb] is not None:
                scat[nb].wait()
                scat[nb] = None
            gath[nb] = pltpu.async_copy(gather_src(jj + 1, nb), rows[nb], gsems[nb])
        gath[b].wait()
        _scale_rows(rows[b], w_bufs[b], 128)
        if scat[b] is not None:
            scat[b].wait()
        scat[b] = pltpu.async_copy(rows[b], scat_dst(jj), ssems[b], add=True)
    scat[0].wait()
    scat[1].wait()


def _prop1_body(srcp, dstp, xp, z2d, z1d,
                ppart, dis_out,
                deg_sh, p_sh,
                swin_v, dwin_v, dis_v, ones_v, w0_v, w1_v, rows0_v, rows1_v,
                gsem0, gsem1, ssem0, ssem1):
    c = lax.axis_index("c")
    s = lax.axis_index("s")
    wid = c * NS + s
    rows = [rows0_v, rows1_v]
    w_bufs = [w0_v, w1_v]
    gsems = [gsem0, gsem1]
    ssems = [ssem0, ssem1]

    pltpu.sync_copy(z2d, p_sh.at[pl.ds(s * NSLICE, NSLICE), :])

    @pl.when(s == 0)
    def _():
        pltpu.sync_copy(z1d, deg_sh)

    for v in range(8):
        ones_v[pl.ds(v * 16, 16)] = jnp.full((16,), 1.0, jnp.float32)
    plsc.subcore_barrier()

    def deg_win(wi, carry):
        pltpu.sync_copy(dstp.at[pl.ds(s * RDEG + wi * 8, 8), :], dwin_v)
        descs = [pltpu.async_copy(ones_v, deg_sh.at[dwin_v.at[jj]], gsem0,
                                  add=True) for jj in range(8)]
        for d in descs:
            d.wait()
        return carry

    lax.fori_loop(0, RDEG // 8, deg_win, 0)
    plsc.subcore_barrier()

    pltpu.sync_copy(deg_sh, dis_v)

    def dis_body(j, carry):
        d = jnp.maximum(dis_v[pl.ds(j * 16, 16)], 1.0)
        dis_v[pl.ds(j * 16, 16)] = _rsqrt_newton(d)
        return carry

    lax.fori_loop(0, NP // 16, dis_body, 0)

    def prop_win(wi, carry):
        pltpu.sync_copy(srcp.at[pl.ds(wid * RPROP + wi * 8, 8), :], swin_v)
        pltpu.sync_copy(dstp.at[pl.ds(wid * RPROP + wi * 8, 8), :], dwin_v)

        def compute_iw(jj, b):
            wb = w_bufs[b]
            for g in range(8):
                sv = swin_v[jj, pl.ds(g * 16, 16)]
                dv = dwin_v[jj, pl.ds(g * 16, 16)]
                wb[pl.ds(g * 16, 16)] = (plsc.load_gather(dis_v, [sv]) *
                                         plsc.load_gather(dis_v, [dv]))

        _pipelined_bursts(compute_iw,
                          lambda jj, b: xp.at[swin_v.at[jj]],
                          rows,
                          lambda jj: p_sh.at[dwin_v.at[jj]],
                          w_bufs, gsems, ssems)
        return carry

    lax.fori_loop(0, RPROP // 8, prop_win, 0)
    plsc.subcore_barrier()

    def wb(chunk, carry):
        base = s * NSLICE + chunk * 128
        pltpu.sync_copy(p_sh.at[pl.ds(base, 128), :], rows0_v)

        def wcopy(i, cc):
            w0_v[pl.ds(i * 16, 16)] = dis_v[pl.ds(base + i * 16, 16)]
            return cc

        lax.fori_loop(0, 8, wcopy, 0)
        _scale_rows(rows0_v, w0_v, 128)
        pltpu.sync_copy(rows0_v, ppart.at[c, pl.ds(base, 128), :])
        return carry

    lax.fori_loop(0, NSLICE // 128, wb, 0)

    @pl.when((s == 0) & (c == 0))
    def _():
        pltpu.sync_copy(dis_v, dis_out)


def _prop1(srcp, dstp, xp):
    z2d = jnp.zeros((NSLICE, 128), jnp.float32)
    z1d = jnp.zeros((NP,), jnp.float32)
    return pl.kernel(
        _prop1_body,
        out_type=(
            jax.ShapeDtypeStruct((NC, NP, 128), jnp.float32),
            jax.ShapeDtypeStruct((NP,), jnp.float32),
        ),
        mesh=_sc_mesh(),
        compiler_params=_SC_PARAMS,
        scratch_types=[
            pltpu.VMEM_SHARED((NP,), jnp.float32),
            pltpu.VMEM_SHARED((NP, 128), jnp.float32),
            pltpu.VMEM((8, 128), jnp.int32),
            pltpu.VMEM((8, 128), jnp.int32),
            pltpu.VMEM((NP,), jnp.float32),
            pltpu.VMEM((128,), jnp.float32),
            pltpu.VMEM((128,), jnp.float32),
            pltpu.VMEM((128,), jnp.float32),
            pltpu.VMEM((128, 128), jnp.float32),
            pltpu.VMEM((128, 128), jnp.float32),
            pltpu.SemaphoreType.DMA,
            pltpu.SemaphoreType.DMA,
            pltpu.SemaphoreType.DMA,
            pltpu.SemaphoreType.DMA,
        ],
    )(srcp, dstp, xp, z2d, z1d)


def _combine_body(srcp, dstp, qb, gd1p, gd2p, z2d,
                  outc,
                  out_sh,
                  swinA_v, dwinA_v, swinB_v, dwinB_v, gdp_v, w0_v, w1_v,
                  idx0_v, idx1_v, rows0_v, rows1_v,
                  gsem0, gsem1, ssem0, ssem1, wsemAs, wsemAd, wsemBs, wsemBd):
    c = lax.axis_index("c")
    s = lax.axis_index("s")
    cbase = c * (N_EXP * NP)
    rows = [rows0_v, rows1_v]
    w_bufs = [w0_v, w1_v]
    idx_bufs = [idx0_v, idx1_v]
    gsems = [gsem0, gsem1]
    ssems = [ssem0, ssem1]
    NW = RDEG // 8

    pltpu.sync_copy(z2d, out_sh.at[pl.ds(s * NSLICE, NSLICE), :])
    plsc.subcore_barrier()

    def issue_win(off, sw, dw, sems, semd):
        pltpu.async_copy(srcp.at[pl.ds(off, 8), :], sw, sems)
        pltpu.async_copy(dstp.at[pl.ds(off, 8), :], dw, semd)

    def wait_win(sw, dw, sems, semd):
        pltpu.make_async_copy(srcp.at[pl.ds(s * RDEG, 8), :], sw, sems).wait()
        pltpu.make_async_copy(dstp.at[pl.ds(s * RDEG, 8), :], dw, semd).wait()

    def process_win(sw, dw):
        def compute_iw(jj, b):
            for g in range(8):
                sl = pl.ds(g * 16, 16)
                sv = sw[jj, sl]
                dv = dw[jj, sl]
                v = plsc.load_gather(gdp_v, [dv])
                ek = v & 7
                w_bufs[b][sl] = lax.bitcast_convert_type(v - ek, jnp.float32)
                idx_bufs[b][sl] = cbase + ek * NP + sv

        _pipelined_bursts(compute_iw,
                          lambda jj, b: qb.at[idx_bufs[b]],
                          rows,
                          lambda jj: out_sh.at[dw.at[jj]],
                          w_bufs, gsems, ssems)

    for k in range(2):
        pltpu.sync_copy(gd1p if k == 0 else gd2p, gdp_v)
        issue_win(s * RDEG, swinA_v, dwinA_v, wsemAs, wsemAd)

        def pair_body(t, carry):
            wait_win(swinA_v, dwinA_v, wsemAs, wsemAd)
            issue_win(s * RDEG + (2 * t + 1) * 8, swinB_v, dwinB_v, wsemBs,
                      wsemBd)
            process_win(swinA_v, dwinA_v)
            offA = s * RDEG + jnp.minimum(2 * t + 2, NW - 1) * 8
            issue_win(offA, swinA_v, dwinA_v, wsemAs, wsemAd)
            wait_win(swinB_v, dwinB_v, wsemBs, wsemBd)
            process_win(swinB_v, dwinB_v)
            return carry

        lax.fori_loop(0, NW // 2, pair_body, 0)
        wait_win(swinA_v, dwinA_v, wsemAs, wsemAd)

    plsc.subcore_barrier()
    pltpu.sync_copy(out_sh.at[pl.ds(s * NSLICE, NSLICE), :],
                    outc.at[c, pl.ds(s * NSLICE, NSLICE), :])


def _combine(srcp, dstp, QB, gd1p, gd2p):
    z2d = jnp.zeros((NSLICE, 128), jnp.float32)
    return pl.kernel(
        _combine_body,
        out_type=jax.ShapeDtypeStruct((NC, NP, 128), jnp.float32),
        mesh=_sc_mesh(),
        compiler_params=_SC_PARAMS,
        scratch_types=[
            pltpu.VMEM_SHARED((NP, 128), jnp.float32),
            pltpu.VMEM((8, 128), jnp.int32),
            pltpu.VMEM((8, 128), jnp.int32),
            pltpu.VMEM((8, 128), jnp.int32),
            pltpu.VMEM((8, 128), jnp.int32),
            pltpu.VMEM((NP,), jnp.int32),
            pltpu.VMEM((128,), jnp.float32),
            pltpu.VMEM((128,), jnp.float32),
            pltpu.VMEM((128,), jnp.int32),
            pltpu.VMEM((128,), jnp.int32),
            pltpu.VMEM((128, 128), jnp.float32),
            pltpu.VMEM((128, 128), jnp.float32),
            pltpu.SemaphoreType.DMA,
            pltpu.SemaphoreType.DMA,
            pltpu.SemaphoreType.DMA,
            pltpu.SemaphoreType.DMA,
            pltpu.SemaphoreType.DMA,
            pltpu.SemaphoreType.DMA,
            pltpu.SemaphoreType.DMA,
            pltpu.SemaphoreType.DMA,
        ],
    )(srcp, dstp, QB, gd1p, gd2p, z2d)


def _gate_body(wgT_ref, xT_ref, dis_ref, gd1_ref, gd2_ref):
    logits = jnp.dot(wgT_ref[...], xT_ref[...], preferred_element_type=jnp.float32)
    m = jnp.max(logits, axis=0, keepdims=True)
    ex = jnp.exp(logits - m)
    gates = ex / jnp.sum(ex, axis=0, keepdims=True)
    iota = lax.broadcasted_iota(jnp.int32, gates.shape, 0)
    v1 = jnp.max(gates, axis=0, keepdims=True)
    i1 = jnp.min(jnp.where(gates == v1, iota, N_EXP), axis=0, keepdims=True)
    masked = jnp.where(iota == i1, -1.0, gates)
    v2 = jnp.max(masked, axis=0, keepdims=True)
    i2 = jnp.min(jnp.where(masked == v2, iota, N_EXP), axis=0, keepdims=True)
    sd = dis_ref[...] / (v1 + v2 + 1e-9)
    b1 = lax.bitcast_convert_type(v1 * sd, jnp.int32)
    b2 = lax.bitcast_convert_type(v2 * sd, jnp.int32)
    gd1_ref[...] = (b1 & ~7) | i1
    gd2_ref[...] = (b2 & ~7) | i2


def _gating(xT_pad, WgT, dis):
    return pl.pallas_call(
        _gate_body,
        out_shape=(
            jax.ShapeDtypeStruct((1, NP), jnp.int32),
            jax.ShapeDtypeStruct((1, NP), jnp.int32),
        ),
    )(WgT, xT_pad, dis)


def _experts_body(p0_ref, p1_ref, w1_ref, w2_ref, q_ref):
    p = p0_ref[...] + p1_ref[...]
    h = jnp.maximum(jnp.dot(p, w1_ref[0], preferred_element_type=jnp.float32), 0.0)
    q = jnp.dot(h, w2_ref[0], preferred_element_type=jnp.float32)
    q_ref[0, 0] = q[:, :128]
    q_ref[1, 0] = q[:, 128:]


def _experts(P0, P1, W1, W2):
    grid = (N_EXP, NP // ROWB)
    return pl.pallas_call(
        _experts_body,
        grid=grid,
        in_specs=[
            pl.BlockSpec((ROWB, D_IN), lambda e, r: (r, 0)),
            pl.BlockSpec((ROWB, D_IN), lambda e, r: (r, 0)),
            pl.BlockSpec((1, D_IN, D_H), lambda e, r: (e, 0, 0)),
            pl.BlockSpec((1, D_H, D_H), lambda e, r: (e, 0, 0)),
        ],
        out_specs=pl.BlockSpec((2, 1, ROWB, 128), lambda e, r: (0, e, r, 0)),
        out_shape=jax.ShapeDtypeStruct((2, N_EXP, NP, 128), jnp.float32),
    )(P0, P1, W1, W2)


def _mlp_body(o0_ref, o1_ref, wa_ref, wb_ref, b1_ref, w2_ref, b2_ref, y_ref):
    h = jnp.dot(o0_ref[...], wa_ref[...], preferred_element_type=jnp.float32)
    h += jnp.dot(o1_ref[...], wb_ref[...], preferred_element_type=jnp.float32)
    h = jnp.maximum(h + b1_ref[...], 0.0)
    y_ref[...] = jnp.dot(h, w2_ref[...], preferred_element_type=jnp.float32) + b2_ref[...]


def _mlp(out0, out1, Wm1, bm1, Wm2, bm2):
    grid = (NP // ROWB,)
    return pl.pallas_call(
        _mlp_body,
        grid=grid,
        in_specs=[
            pl.BlockSpec((ROWB, 128), lambda r: (r, 0)),
            pl.BlockSpec((ROWB, 128), lambda r: (r, 0)),
            pl.BlockSpec((128, D_H), lambda r: (0, 0)),
            pl.BlockSpec((128, D_H), lambda r: (0, 0)),
            pl.BlockSpec((1, D_H), lambda r: (0, 0)),
            pl.BlockSpec((D_H, D_OUT), lambda r: (0, 0)),
            pl.BlockSpec((1, D_OUT), lambda r: (0, 0)),
        ],
        out_specs=pl.BlockSpec((ROWB, D_OUT), lambda r: (r, 0)),
        out_shape=jax.ShapeDtypeStruct((NP, D_OUT), jnp.float32),
    )(out0, out1, Wm1[:128], Wm1[128:], bm1[None, :], Wm2, bm2[None, :])


def kernel(x, edge_index, batch, Wg, W1, W2, Wm1, bm1, Wm2, bm2):
    src = edge_index[0]
    dst = edge_index[1]
    xp = jnp.pad(x, ((0, NP - N), (0, 0)))

    pad = jnp.full((E_PAD - E,), NP - 1, jnp.int32)
    srcp = jnp.concatenate([src, pad]).reshape(E_PAD // 128, 128)
    dstp = jnp.concatenate([dst, pad]).reshape(E_PAD // 128, 128)

    Ppart, dis = _prop1(srcp, dstp, xp)

    gd1p, gd2p = _gating(xp.T, Wg.T, dis[None, :])

    QB = _experts(Ppart[0], Ppart[1], W1, W2).reshape(2 * N_EXP * NP, 128)

    outc = _combine(srcp, dstp, QB, gd1p[0], gd2p[0])

    y = _mlp(outc[0], outc[1], Wm1, bm1, Wm2, bm2)
    return y[:N]


# speedup vs baseline: 1.4080x; 1.0123x over previous
import functools

import jax
import jax.numpy as jnp
from jax import lax
from jax.experimental import pallas as pl
from jax.experimental.pallas import tpu as pltpu
from jax.experimental.pallas import tpu_sc as plsc

N = 10000
NP = 10240
E = 320000
E_PAD = 327680
D_IN = 128
D_H = 256
D_OUT = 64
N_EXP = 8
ROWB = 1024
NC, NS = 2, 16
RDEG = E_PAD // NS // 128
RPROP = E_PAD // (NC * NS) // 128
NSLICE = NP // NS


def _sc_mesh():
    return plsc.VectorSubcoreMesh(
        core_axis_name="c", subcore_axis_name="s", num_cores=NC, num_subcores=NS)


_SC_PARAMS = pltpu.CompilerParams(needs_layout_passes=False)


def _rsqrt_newton(d):
    i = lax.bitcast_convert_type(d, jnp.int32)
    i = 0x5F3759DF - lax.shift_right_logical(i, 1)
    y = lax.bitcast_convert_type(i, jnp.float32)
    for _ in range(3):
        y = y * (1.5 - 0.5 * d * y * y)
    return y


def _scale_rows(rows_v, w_v, nrows):
    def body(r4, cc):
        ra = r4 * 4
        wvs = [plsc.load_gather(w_v, [jnp.zeros((16,), jnp.int32) + (ra + u)])
               for u in range(4)]
        for v in range(8):
            sl = pl.ds(v * 16, 16)
            for u in range(4):
                rows_v[ra + u, sl] = rows_v[ra + u, sl] * wvs[u]
        return cc

    lax.fori_loop(0, nrows // 4, body, 0)


def _pipelined_bursts(compute_iw, gather_src, rows, scat_dst, w_bufs, gsems, ssems):
    scat = [None, None]
    gath = [None, None]
    compute_iw(0, 0)
    gath[0] = pltpu.async_copy(gather_src(0, 0), rows[0], gsems[0])
    for jj in range(8):
        b = jj % 2
        nb = (jj + 1) % 2
        if jj < 7:
            compute_iw(jj + 1, nb)
            if scat[nb] is not None:
                scat[nb].wait()
                scat[nb] = None
            gath[nb] = pltpu.async_copy(gather_src(jj + 1, nb), rows[nb], gsems[nb])
        gath[b].wait()
        _scale_rows(rows[b], w_bufs[b], 128)
        if scat[b] is not None:
            scat[b].wait()
        scat[b] = pltpu.async_copy(rows[b], scat_dst(jj), ssems[b], add=True)
    scat[0].wait()
    scat[1].wait()


def _prop1_body(srcp, dstp, xp, z2d, z1d,
                ppart, dis_out,
                deg_sh, p_sh,
                swin_v, dwin_v, dis_v, ones_v, w0_v, w1_v, rows0_v, rows1_v,
                gsem0, gsem1, ssem0, ssem1):
    c = lax.axis_index("c")
    s = lax.axis_index("s")
    wid = c * NS + s
    rows = [rows0_v, rows1_v]
    w_bufs = [w0_v, w1_v]
    gsems = [gsem0, gsem1]
    ssems = [ssem0, ssem1]

    pltpu.sync_copy(z2d, p_sh.at[pl.ds(s * NSLICE, NSLICE), :])

    @pl.when(s == 0)
    def _():
        pltpu.sync_copy(z1d, deg_sh)

    for v in range(8):
        ones_v[pl.ds(v * 16, 16)] = jnp.full((16,), 1.0, jnp.float32)
    plsc.subcore_barrier()

    def deg_win(wi, carry):
        pltpu.sync_copy(dstp.at[pl.ds(s * RDEG + wi * 8, 8), :], dwin_v)
        descs = [pltpu.async_copy(ones_v, deg_sh.at[dwin_v.at[jj]], gsem0,
                                  add=True) for jj in range(8)]
        for d in descs:
            d.wait()
        return carry

    lax.fori_loop(0, RDEG // 8, deg_win, 0)
    plsc.subcore_barrier()

    pltpu.sync_copy(deg_sh, dis_v)

    def dis_body(j, carry):
        d = jnp.maximum(dis_v[pl.ds(j * 16, 16)], 1.0)
        dis_v[pl.ds(j * 16, 16)] = _rsqrt_newton(d)
        return carry

    lax.fori_loop(0, NP // 16, dis_body, 0)

    def prop_win(wi, carry):
        pltpu.sync_copy(srcp.at[pl.ds(wid * RPROP + wi * 8, 8), :], swin_v)
        pltpu.sync_copy(dstp.at[pl.ds(wid * RPROP + wi * 8, 8), :], dwin_v)

        def compute_iw(jj, b):
            wb = w_bufs[b]
            for g in range(8):
                sv = swin_v[jj, pl.ds(g * 16, 16)]
                dv = dwin_v[jj, pl.ds(g * 16, 16)]
                wb[pl.ds(g * 16, 16)] = (plsc.load_gather(dis_v, [sv]) *
                                         plsc.load_gather(dis_v, [dv]))

        _pipelined_bursts(compute_iw,
                          lambda jj, b: xp.at[swin_v.at[jj]],
                          rows,
                          lambda jj: p_sh.at[dwin_v.at[jj]],
                          w_bufs, gsems, ssems)
        return carry

    lax.fori_loop(0, RPROP // 8, prop_win, 0)
    plsc.subcore_barrier()

    def wb(chunk, carry):
        base = s * NSLICE + chunk * 128
        pltpu.sync_copy(p_sh.at[pl.ds(base, 128), :], rows0_v)

        def wcopy(i, cc):
            w0_v[pl.ds(i * 16, 16)] = dis_v[pl.ds(base + i * 16, 16)]
            return cc

        lax.fori_loop(0, 8, wcopy, 0)
        _scale_rows(rows0_v, w0_v, 128)
        pltpu.sync_copy(rows0_v, ppart.at[c, pl.ds(base, 128), :])
        return carry

    lax.fori_loop(0, NSLICE // 128, wb, 0)

    @pl.when((s == 0) & (c == 0))
    def _():
        pltpu.sync_copy(dis_v, dis_out)


def _prop1(srcp, dstp, xp):
    z2d = jnp.zeros((NSLICE, 128), jnp.float32)
    z1d = jnp.zeros((NP,), jnp.float32)
    return pl.kernel(
        _prop1_body,
        out_type=(
            jax.ShapeDtypeStruct((NC, NP, 128), jnp.float32),
            jax.ShapeDtypeStruct((NP,), jnp.float32),
        ),
        mesh=_sc_mesh(),
        compiler_params=_SC_PARAMS,
        scratch_types=[
            pltpu.VMEM_SHARED((NP,), jnp.float32),
            pltpu.VMEM_SHARED((NP, 128), jnp.float32),
            pltpu.VMEM((8, 128), jnp.int32),
            pltpu.VMEM((8, 128), jnp.int32),
            pltpu.VMEM((NP,), jnp.float32),
            pltpu.VMEM((128,), jnp.float32),
            pltpu.VMEM((128,), jnp.float32),
            pltpu.VMEM((128,), jnp.float32),
            pltpu.VMEM((128, 128), jnp.float32),
            pltpu.VMEM((128, 128), jnp.float32),
            pltpu.SemaphoreType.DMA,
            pltpu.SemaphoreType.DMA,
            pltpu.SemaphoreType.DMA,
            pltpu.SemaphoreType.DMA,
        ],
    )(srcp, dstp, xp, z2d, z1d)


def _combine_body(srcp, dstp, qb, gd1p, gd2p, z2d,
                  outc,
                  out_sh,
                  swinA_v, dwinA_v, swinB_v, dwinB_v, gdp_v, w0_v, w1_v,
                  idx0_v, idx1_v, rows0_v, rows1_v,
                  gsem0, gsem1, ssem0, ssem1, wsemAs, wsemAd, wsemBs, wsemBd):
    c = lax.axis_index("c")
    s = lax.axis_index("s")
    cbase = c * (N_EXP * NP)
    rows = [rows0_v, rows1_v]
    w_bufs = [w0_v, w1_v]
    idx_bufs = [idx0_v, idx1_v]
    gsems = [gsem0, gsem1]
    ssems = [ssem0, ssem1]
    NW = RDEG // 8

    pltpu.sync_copy(z2d, out_sh.at[pl.ds(s * NSLICE, NSLICE), :])
    plsc.subcore_barrier()

    def issue_win(off, sw, dw, sems, semd):
        pltpu.async_copy(srcp.at[pl.ds(off, 8), :], sw, sems)
        pltpu.async_copy(dstp.at[pl.ds(off, 8), :], dw, semd)

    def wait_win(sw, dw, sems, semd):
        pltpu.make_async_copy(srcp.at[pl.ds(s * RDEG, 8), :], sw, sems).wait()
        pltpu.make_async_copy(dstp.at[pl.ds(s * RDEG, 8), :], dw, semd).wait()

    def process_win(sw, dw):
        def compute_iw(jj, b):
            for g in range(8):
                sl = pl.ds(g * 16, 16)
                sv = sw[jj, sl]
                dv = dw[jj, sl]
                v = plsc.load_gather(gdp_v, [dv])
                ek = v & 7
                w_bufs[b][sl] = lax.bitcast_convert_type(v - ek, jnp.float32)
                idx_bufs[b][sl] = cbase + ek * NP + sv

        _pipelined_bursts(compute_iw,
                          lambda jj, b: qb.at[idx_bufs[b]],
                          rows,
                          lambda jj: out_sh.at[dw.at[jj]],
                          w_bufs, gsems, ssems)

    for k in range(2):
        pltpu.sync_copy(gd1p if k == 0 else gd2p, gdp_v)
        issue_win(s * RDEG, swinA_v, dwinA_v, wsemAs, wsemAd)

        def pair_body(t, carry):
            wait_win(swinA_v, dwinA_v, wsemAs, wsemAd)
            issue_win(s * RDEG + (2 * t + 1) * 8, swinB_v, dwinB_v, wsemBs,
                      wsemBd)
            process_win(swinA_v, dwinA_v)
            offA = s * RDEG + jnp.minimum(2 * t + 2, NW - 1) * 8
            issue_win(offA, swinA_v, dwinA_v, wsemAs, wsemAd)
            wait_win(swinB_v, dwinB_v, wsemBs, wsemBd)
            process_win(swinB_v, dwinB_v)
            return carry

        lax.fori_loop(0, NW // 2, pair_body, 0)
        wait_win(swinA_v, dwinA_v, wsemAs, wsemAd)

    plsc.subcore_barrier()
    pltpu.sync_copy(out_sh.at[pl.ds(s * NSLICE, NSLICE), :],
                    outc.at[c, pl.ds(s * NSLICE, NSLICE), :])


def _combine(srcp, dstp, QB, gd1p, gd2p):
    z2d = jnp.zeros((NSLICE, 128), jnp.float32)
    return pl.kernel(
        _combine_body,
        out_type=jax.ShapeDtypeStruct((NC, NP, 128), jnp.float32),
        mesh=_sc_mesh(),
        compiler_params=_SC_PARAMS,
        scratch_types=[
            pltpu.VMEM_SHARED((NP, 128), jnp.float32),
            pltpu.VMEM((8, 128), jnp.int32),
            pltpu.VMEM((8, 128), jnp.int32),
            pltpu.VMEM((8, 128), jnp.int32),
            pltpu.VMEM((8, 128), jnp.int32),
            pltpu.VMEM((NP,), jnp.int32),
            pltpu.VMEM((128,), jnp.float32),
            pltpu.VMEM((128,), jnp.float32),
            pltpu.VMEM((128,), jnp.int32),
            pltpu.VMEM((128,), jnp.int32),
            pltpu.VMEM((128, 128), jnp.float32),
            pltpu.VMEM((128, 128), jnp.float32),
            pltpu.SemaphoreType.DMA,
            pltpu.SemaphoreType.DMA,
            pltpu.SemaphoreType.DMA,
            pltpu.SemaphoreType.DMA,
            pltpu.SemaphoreType.DMA,
            pltpu.SemaphoreType.DMA,
            pltpu.SemaphoreType.DMA,
            pltpu.SemaphoreType.DMA,
        ],
    )(srcp, dstp, QB, gd1p, gd2p, z2d)


def _gate_body(wgT_ref, xT_ref, dis_ref, gd1_ref, gd2_ref):
    logits = jnp.dot(wgT_ref[...], xT_ref[...], preferred_element_type=jnp.float32)
    m = jnp.max(logits, axis=0, keepdims=True)
    ex = jnp.exp(logits - m)
    gates = ex / jnp.sum(ex, axis=0, keepdims=True)
    iota = lax.broadcasted_iota(jnp.int32, gates.shape, 0)
    v1 = jnp.max(gates, axis=0, keepdims=True)
    i1 = jnp.min(jnp.where(gates == v1, iota, N_EXP), axis=0, keepdims=True)
    masked = jnp.where(iota == i1, -1.0, gates)
    v2 = jnp.max(masked, axis=0, keepdims=True)
    i2 = jnp.min(jnp.where(masked == v2, iota, N_EXP), axis=0, keepdims=True)
    sd = dis_ref[...] / (v1 + v2 + 1e-9)
    b1 = lax.bitcast_convert_type(v1 * sd, jnp.int32)
    b2 = lax.bitcast_convert_type(v2 * sd, jnp.int32)
    gd1_ref[...] = (b1 & ~7) | i1
    gd2_ref[...] = (b2 & ~7) | i2


def _gating(xT_pad, WgT, dis):
    return pl.pallas_call(
        _gate_body,
        out_shape=(
            jax.ShapeDtypeStruct((1, NP), jnp.int32),
            jax.ShapeDtypeStruct((1, NP), jnp.int32),
        ),
    )(WgT, xT_pad, dis)


def _experts_body(p0_ref, p1_ref, w1_ref, w2_ref, q_ref):
    p = p0_ref[...] + p1_ref[...]
    h = jnp.maximum(jnp.dot(p, w1_ref[0], preferred_element_type=jnp.float32), 0.0)
    q = jnp.dot(h, w2_ref[0], preferred_element_type=jnp.float32)
    q_ref[0, 0] = q[:, :128]
    q_ref[1, 0] = q[:, 128:]


def _experts(P0, P1, W1, W2):
    grid = (N_EXP, NP // ROWB)
    return pl.pallas_call(
        _experts_body,
        grid=grid,
        in_specs=[
            pl.BlockSpec((ROWB, D_IN), lambda e, r: (r, 0)),
            pl.BlockSpec((ROWB, D_IN), lambda e, r: (r, 0)),
            pl.BlockSpec((1, D_IN, D_H), lambda e, r: (e, 0, 0)),
            pl.BlockSpec((1, D_H, D_H), lambda e, r: (e, 0, 0)),
        ],
        out_specs=pl.BlockSpec((2, 1, ROWB, 128), lambda e, r: (0, e, r, 0)),
        out_shape=jax.ShapeDtypeStruct((2, N_EXP, NP, 128), jnp.float32),
    )(P0, P1, W1, W2)


def _mlp_body(o0_ref, o1_ref, wa_ref, wb_ref, b1_ref, w2_ref, b2_ref, y_ref):
    h = jnp.dot(o0_ref[...], wa_ref[...], preferred_element_type=jnp.float32)
    h += jnp.dot(o1_ref[...], wb_ref[...], preferred_element_type=jnp.float32)
    h = jnp.maximum(h + b1_ref[...], 0.0)
    y_ref[...] = jnp.dot(h, w2_ref[...], preferred_element_type=jnp.float32) + b2_ref[...]


def _mlp(out0, out1, Wm1, bm1, Wm2, bm2):
    grid = (NP // ROWB,)
    return pl.pallas_call(
        _mlp_body,
        grid=grid,
        in_specs=[
            pl.BlockSpec((ROWB, 128), lambda r: (r, 0)),
            pl.BlockSpec((ROWB, 128), lambda r: (r, 0)),
            pl.BlockSpec((128, D_H), lambda r: (0, 0)),
            pl.BlockSpec((128, D_H), lambda r: (0, 0)),
            pl.BlockSpec((1, D_H), lambda r: (0, 0)),
            pl.BlockSpec((D_H, D_OUT), lambda r: (0, 0)),
            pl.BlockSpec((1, D_OUT), lambda r: (0, 0)),
        ],
        out_specs=pl.BlockSpec((ROWB, D_OUT), lambda r: (r, 0)),
        out_shape=jax.ShapeDtypeStruct((NP, D_OUT), jnp.float32),
    )(out0, out1, Wm1[:128], Wm1[128:], bm1[None, :], Wm2, bm2[None, :])


def kernel(x, edge_index, batch, Wg, W1, W2, Wm1, bm1, Wm2, bm2):
    src = edge_index[0]
    dst = edge_index[1]
    xp = jnp.pad(x, ((0, NP - N), (0, 0)))

    pad = jnp.full((E_PAD - E,), NP - 1, jnp.int32)
    srcp = jnp.concatenate([src, pad]).reshape(E_PAD // 128, 128)
    dstp = jnp.concatenate([dst, pad]).reshape(E_PAD // 128, 128)

    Ppart, dis = _prop1(srcp, dstp, xp)

    gd1p, gd2p = _gating(xp.T, Wg.T, dis[None, :])

    QB = _experts(Ppart[0], Ppart[1], W1, W2).reshape(2 * N_EXP * NP, 128)

    outc = _combine(srcp, dstp, QB, gd1p[0], gd2p[0])

    y = _mlp(outc[0], outc[1], Wm1, bm1, Wm2, bm2)
    return y[:N]
